# baseline (speedup 1.0000x reference)
import jax
import jax.numpy as jnp
from jax import lax
from jax.experimental import pallas as pl
from jax.experimental.pallas import tpu as pltpu

_N = 32768
_D = 768
_E = 8
_B = 1024


def _moe_body(gf_ref, x_ref, t_ref, wg_ref, es_ref, eb_ref,
              misc_ref, loss_ref, acc_ref):
    i = pl.program_id(0)

    @pl.when(i == 0)
    def _init():
        acc_ref[...] = jnp.zeros_like(acc_ref)

    logits = jnp.dot(gf_ref[...], wg_ref[...], preferred_element_type=jnp.float32)
    logits_t = jnp.transpose(logits)
    m_t = jnp.max(logits_t, axis=0, keepdims=True)
    ex_t = jnp.exp(logits_t - m_t)
    sum_t = jnp.sum(ex_t, axis=0, keepdims=True)
    probs_t = ex_t / sum_t
    pmax_t = jnp.max(probs_t, axis=0, keepdims=True)

    iota_t = lax.broadcasted_iota(jnp.int32, probs_t.shape, 0).astype(jnp.float32)
    assign_t = jnp.min(jnp.where(probs_t == pmax_t, iota_t, float(_E)),
                       axis=0, keepdims=True)
    oh_t = (iota_t == assign_t).astype(jnp.float32)
    oh = jnp.transpose(oh_t)

    scale = jnp.dot(oh, es_ref[...], preferred_element_type=jnp.float32)
    bias = jnp.dot(oh, eb_ref[...], preferred_element_type=jnp.float32)
    diff = x_ref[...] * scale + bias - t_ref[...]
    per_tok = jnp.sum(diff * diff, axis=1, keepdims=True) * (1.0 / _D)

    pt2 = jnp.concatenate([per_tok, jnp.ones_like(per_tok)], axis=1)
    acc_ref[...] += jnp.dot(oh_t, pt2, preferred_element_type=jnp.float32)

    misc_ref[...] = jnp.concatenate([probs_t, pmax_t, assign_t], axis=0)

    @pl.when(i == pl.num_programs(0) - 1)
    def _fini():
        s = acc_ref[:, 0:1]
        c = acc_ref[:, 1:2]
        loss_ref[...] = jnp.sum(s / jnp.maximum(c, 1.0)).reshape(1, 1)


def _run(gate_features, x, target, Wg, expert_scale, expert_bias):
    grid = _N // _B
    misc, loss = pl.pallas_call(
        _moe_body,
        grid=(grid,),
        in_specs=[
            pl.BlockSpec((_B, _D), lambda i: (i, 0)),
            pl.BlockSpec((_B, _D), lambda i: (i, 0)),
            pl.BlockSpec((_B, _D), lambda i: (i, 0)),
            pl.BlockSpec((_D, _E), lambda i: (0, 0)),
            pl.BlockSpec((_E, _D), lambda i: (0, 0)),
            pl.BlockSpec((_E, _D), lambda i: (0, 0)),
        ],
        out_specs=[
            pl.BlockSpec((_E + 2, _B), lambda i: (0, i)),
            pl.BlockSpec((1, 1), lambda i: (0, 0)),
        ],
        out_shape=[
            jax.ShapeDtypeStruct((_E + 2, _N), jnp.float32),
            jax.ShapeDtypeStruct((1, 1), jnp.float32),
        ],
        scratch_shapes=[
            pltpu.VMEM((_E, 2), jnp.float32),
        ],
    )(gate_features, x, target, Wg, expert_scale, expert_bias)
    return misc, loss


def kernel(gate_features, x, target, Wg, expert_scale, expert_bias):
    misc, loss = _run(gate_features, x, target, Wg, expert_scale, expert_bias)
    total_loss = loss[0, 0]
    probs = misc[:_E].T
    topk_probs = misc[_E:_E + 1].T
    assignments = misc[_E + 1].astype(jnp.int32)
    topk_idx = assignments[:, None]
    return (total_loss, assignments, probs, topk_idx, topk_probs)


# speedup vs baseline: 1.0963x; 1.0963x over previous
import jax
import jax.numpy as jnp
from jax import lax
from jax.experimental import pallas as pl
from jax.experimental.pallas import tpu as pltpu

_N = 32768
_D = 768
_E = 8
_B = 2048


def _moe_body(gf_ref, x_ref, t_ref, wg_ref, es_ref, eb_ref,
              misc_ref, loss_ref, acc_ref):
    i = pl.program_id(0)

    @pl.when(i == 0)
    def _init():
        acc_ref[...] = jnp.zeros_like(acc_ref)

    logits = jnp.dot(gf_ref[...], wg_ref[...], preferred_element_type=jnp.float32)
    logits_t = jnp.transpose(logits)
    m_t = jnp.max(logits_t, axis=0, keepdims=True)
    ex_t = jnp.exp(logits_t - m_t)
    sum_t = jnp.sum(ex_t, axis=0, keepdims=True)
    probs_t = ex_t / sum_t
    pmax_t = jnp.max(probs_t, axis=0, keepdims=True)

    iota_t = lax.broadcasted_iota(jnp.int32, probs_t.shape, 0).astype(jnp.float32)
    assign_t = jnp.min(jnp.where(probs_t == pmax_t, iota_t, float(_E)),
                       axis=0, keepdims=True)
    oh_t = (iota_t == assign_t).astype(jnp.float32)
    oh = jnp.transpose(oh_t)

    scale = jnp.dot(oh, es_ref[...], preferred_element_type=jnp.float32)
    bias = jnp.dot(oh, eb_ref[...], preferred_element_type=jnp.float32)
    diff = x_ref[...] * scale + bias - t_ref[...]
    per_tok = jnp.sum(diff * diff, axis=1, keepdims=True) * (1.0 / _D)

    pt2 = jnp.concatenate([per_tok, jnp.ones_like(per_tok)], axis=1)
    acc_ref[...] += jnp.dot(oh_t, pt2, preferred_element_type=jnp.float32)

    misc_ref[...] = jnp.concatenate([probs_t, pmax_t, assign_t], axis=0)

    @pl.when(i == pl.num_programs(0) - 1)
    def _fini():
        s = acc_ref[:, 0:1]
        c = acc_ref[:, 1:2]
        loss_ref[...] = jnp.sum(s / jnp.maximum(c, 1.0)).reshape(1, 1)


def _run(gate_features, x, target, Wg, expert_scale, expert_bias):
    grid = _N // _B
    misc, loss = pl.pallas_call(
        _moe_body,
        grid=(grid,),
        in_specs=[
            pl.BlockSpec((_B, _D), lambda i: (i, 0)),
            pl.BlockSpec((_B, _D), lambda i: (i, 0)),
            pl.BlockSpec((_B, _D), lambda i: (i, 0)),
            pl.BlockSpec((_D, _E), lambda i: (0, 0)),
            pl.BlockSpec((_E, _D), lambda i: (0, 0)),
            pl.BlockSpec((_E, _D), lambda i: (0, 0)),
        ],
        out_specs=[
            pl.BlockSpec((_E + 2, _B), lambda i: (0, i)),
            pl.BlockSpec((1, 1), lambda i: (0, 0)),
        ],
        out_shape=[
            jax.ShapeDtypeStruct((_E + 2, _N), jnp.float32),
            jax.ShapeDtypeStruct((1, 1), jnp.float32),
        ],
        scratch_shapes=[
            pltpu.VMEM((_E, 2), jnp.float32),
        ],
    )(gate_features, x, target, Wg, expert_scale, expert_bias)
    return misc, loss


def kernel(gate_features, x, target, Wg, expert_scale, expert_bias):
    misc, loss = _run(gate_features, x, target, Wg, expert_scale, expert_bias)
    total_loss = loss[0, 0]
    probs = misc[:_E].T
    topk_probs = misc[_E:_E + 1].T
    assignments = misc[_E + 1].astype(jnp.int32)
    topk_idx = assignments[:, None]
    return (total_loss, assignments, probs, topk_idx, topk_probs)
